# baseline (speedup 1.0000x reference)
import functools

import jax
import jax.numpy as jnp
from jax import lax
from jax.experimental import pallas as pl
from jax.experimental.pallas import tpu as pltpu
from jax.experimental.pallas import tpu_sc as plsc

N_PW = 8
EMBED = 128
L = 16
NC, NS = 2, 16
NW = NC * NS
C = 128
WTM_ROWS = 560
PAIR_ROWS = 2401
PAIR_PAD = 2432
TBL_ROWS = WTM_ROWS + 4 * PAIR_PAD


def _build_table(W, b):
    WT = W.T
    w_ = jnp.arange(8)
    tl_ = jnp.arange(10)
    mtl_ = jnp.arange(7)
    t_wtm = (WT[w_ + 1][:, None, None, :]
             + WT[9 + tl_][None, :, None, :]
             + WT[19 + mtl_][None, None, :, :]
             + b).reshape(WTM_ROWS, EMBED)
    v = jnp.arange(7)
    parts = [t_wtm]
    for j in range(4):
        p0, p1 = 2 * j, 2 * j + 1
        mn0 = WT[106 + 8 * p0 + v + 1]
        mx0 = WT[26 + 10 * p0 + v + 1]
        mn1 = WT[106 + 8 * p1 + v + 1]
        mx1 = WT[26 + 10 * p1 + v + 1]
        tj = (mn0[:, None, None, None, :] + mx0[None, :, None, None, :]
              + mn1[None, None, :, None, :] + mx1[None, None, None, :, :]
              ).reshape(PAIR_ROWS, EMBED)
        parts.append(jnp.pad(tj, ((0, PAIR_PAD - PAIR_ROWS), (0, 0))))
    return jnp.concatenate(parts, axis=0)


def _sc_body(tbl, w_h, tl_h, mtl_h, pw_h, out_h,
             w_v, tl_v, mtl_v, pw_v, idx_v, rows_v, out_v, sem):
    wid = lax.axis_index("s") * NC + lax.axis_index("c")
    per_w = out_h.shape[0] // NW
    lanes = lax.iota(jnp.int32, L)

    def chunk(k, carry):
        base = wid * per_w + k * C
        pltpu.sync_copy(w_h.at[pl.ds(base, C)], w_v)
        pltpu.sync_copy(tl_h.at[pl.ds(base, C)], tl_v)
        pltpu.sync_copy(mtl_h.at[pl.ds(base, C)], mtl_v)
        pltpu.sync_copy(pw_h.at[pl.ds(base * (2 * N_PW), C * 2 * N_PW)], pw_v)
        for g in range(C // L):
            sl = pl.ds(g * L, L)
            idx_v[0, sl] = (w_v[sl] * 10 + tl_v[sl]) * 7 + mtl_v[sl]
            flat16 = (lanes + g * L) * (2 * N_PW)
            for j in range(4):
                cols = [plsc.load_gather(pw_v, [flat16 + (4 * j + t)])
                        for t in range(4)]
                m0, x0, m1, x1 = cols
                idx_v[1 + j, sl] = (((m0 * 7 + x0) * 7 + m1) * 7 + x1
                                    + (WTM_ROWS + PAIR_PAD * j))
        cps = [pltpu.async_copy(tbl.at[idx_v.at[s]], rows_v.at[s], sem)
               for s in range(5)]
        for cp in cps:
            cp.wait()

        def acc_row(c, carry2):
            for vv in range(EMBED // L):
                slv = pl.ds(vv * L, L)
                a = rows_v[0, c, slv] + rows_v[1, c, slv]
                a = a + rows_v[2, c, slv]
                a = a + rows_v[3, c, slv]
                a = a + rows_v[4, c, slv]
                out_v[c, slv] = a
            return carry2

        lax.fori_loop(0, C, acc_row, 0)
        pltpu.sync_copy(out_v, out_h.at[pl.ds(base, C)])
        return carry

    lax.fori_loop(0, out_h.shape[0] // NW // C, chunk, 0)


def kernel(weather, time_left, min_time_left, pseudo_weather, W, b):
    B, T = weather.shape
    N = B * T
    tbl = _build_table(W, b)
    w_f = weather.reshape(N).astype(jnp.int32)
    tl_f = time_left.reshape(N).astype(jnp.int32)
    mtl_f = min_time_left.reshape(N).astype(jnp.int32)
    pw_f = pseudo_weather.reshape(N * 2 * N_PW).astype(jnp.int32)

    mesh = plsc.VectorSubcoreMesh(core_axis_name="c", subcore_axis_name="s")
    run = pl.kernel(
        _sc_body,
        out_type=jax.ShapeDtypeStruct((N, EMBED), jnp.float32),
        mesh=mesh,
        compiler_params=pltpu.CompilerParams(needs_layout_passes=False),
        scratch_types=[
            pltpu.VMEM((C,), jnp.int32),
            pltpu.VMEM((C,), jnp.int32),
            pltpu.VMEM((C,), jnp.int32),
            pltpu.VMEM((C * 2 * N_PW,), jnp.int32),
            pltpu.VMEM((5, C), jnp.int32),
            pltpu.VMEM((5, C, EMBED), jnp.float32),
            pltpu.VMEM((C, EMBED), jnp.float32),
            pltpu.SemaphoreType.DMA,
        ],
    )
    out = run(tbl, w_f, tl_f, mtl_f, pw_f)
    return out.reshape(B, T, EMBED)


# speedup vs baseline: 5.5381x; 5.5381x over previous
import functools

import jax
import jax.numpy as jnp
from jax import lax
from jax.experimental import pallas as pl
from jax.experimental.pallas import tpu as pltpu
from jax.experimental.pallas import tpu_sc as plsc

N_PW = 8
EMBED = 128
L = 16
NC, NS = 2, 16
NW = NC * NS
C = 128
NINT = 3 + 2 * N_PW
WTM_ROWS = 560
PAIR_ROWS = 2401
PAIR_PAD = 2432
TBL_ROWS = WTM_ROWS + 4 * PAIR_PAD


def _build_table(W, b):
    WT = W.T
    w_ = jnp.arange(8)
    tl_ = jnp.arange(10)
    mtl_ = jnp.arange(7)
    t_wtm = (WT[w_ + 1][:, None, None, :]
             + WT[9 + tl_][None, :, None, :]
             + WT[19 + mtl_][None, None, :, :]
             + b).reshape(WTM_ROWS, EMBED)
    v = jnp.arange(7)
    parts = [t_wtm]
    for j in range(4):
        p0, p1 = 2 * j, 2 * j + 1
        mn0 = WT[106 + 8 * p0 + v + 1]
        mx0 = WT[26 + 10 * p0 + v + 1]
        mn1 = WT[106 + 8 * p1 + v + 1]
        mx1 = WT[26 + 10 * p1 + v + 1]
        tj = (mn0[:, None, None, None, :] + mx0[None, :, None, None, :]
              + mn1[None, None, :, None, :] + mx1[None, None, None, :, :]
              ).reshape(PAIR_ROWS, EMBED)
        parts.append(jnp.pad(tj, ((0, PAIR_PAD - PAIR_ROWS), (0, 0))))
    return jnp.concatenate(parts, axis=0)


def _sc_body(tbl, ints_h, out_h, iv, idx_v, rows_v, out_v, sem):
    wid = lax.axis_index("s") * NC + lax.axis_index("c")
    n_t = out_h.shape[0]
    b0 = wid * C

    def step(t, carry):
        pltpu.sync_copy(ints_h.at[t, :, pl.ds(b0, C)], iv)
        for g in range(C // L):
            sl = pl.ds(g * L, L)
            idx_v[0, sl] = (iv[0, sl] * 10 + iv[1, sl]) * 7 + iv[2, sl]
            for j in range(4):
                m0 = iv[3 + 4 * j, sl]
                x0 = iv[4 + 4 * j, sl]
                m1 = iv[5 + 4 * j, sl]
                x1 = iv[6 + 4 * j, sl]
                idx_v[1 + j, sl] = (((m0 * 7 + x0) * 7 + m1) * 7 + x1
                                    + (WTM_ROWS + PAIR_PAD * j))
        cps = [pltpu.async_copy(tbl.at[idx_v.at[s]], rows_v.at[s], sem)
               for s in range(5)]
        for cp in cps:
            cp.wait()

        def acc_row(c, carry2):
            for vv in range(EMBED // L):
                slv = pl.ds(vv * L, L)
                a = rows_v[0, c, slv] + rows_v[1, c, slv]
                a = a + rows_v[2, c, slv]
                a = a + rows_v[3, c, slv]
                a = a + rows_v[4, c, slv]
                out_v[c, slv] = a
            return carry2

        lax.fori_loop(0, C, acc_row, 0)
        pltpu.sync_copy(out_v, out_h.at[t, pl.ds(b0, C)])
        return carry

    lax.fori_loop(0, n_t, step, 0)


def kernel(weather, time_left, min_time_left, pseudo_weather, W, b):
    B, T = weather.shape
    tbl = _build_table(W, b)
    ints_t = jnp.concatenate(
        [weather.T.astype(jnp.int32)[:, None, :],
         time_left.T.astype(jnp.int32)[:, None, :],
         min_time_left.T.astype(jnp.int32)[:, None, :],
         jnp.transpose(pseudo_weather.astype(jnp.int32),
                       (1, 2, 3, 0)).reshape(T, 2 * N_PW, B)],
        axis=1)

    mesh = plsc.VectorSubcoreMesh(core_axis_name="c", subcore_axis_name="s")
    run = pl.kernel(
        _sc_body,
        out_type=jax.ShapeDtypeStruct((T, B, EMBED), jnp.float32),
        mesh=mesh,
        compiler_params=pltpu.CompilerParams(needs_layout_passes=False),
        scratch_types=[
            pltpu.VMEM((NINT, C), jnp.int32),
            pltpu.VMEM((5, C), jnp.int32),
            pltpu.VMEM((5, C, EMBED), jnp.float32),
            pltpu.VMEM((C, EMBED), jnp.float32),
            pltpu.SemaphoreType.DMA,
        ],
    )
    out = run(tbl, ints_t)
    return jnp.transpose(out, (1, 0, 2))
